# Initial kernel scaffold; baseline (speedup 1.0000x reference)
import jax
import jax.numpy as jnp
from jax.experimental import pallas as pl


def kernel(x, edge_index, W_t1, b_t1, W_t2, b_t2, W_c1, b_c1, W_c2, b_c2, matrix, W_lin, b_lin):
    raise NotImplementedError("write your pallas kernel here")


# speedup vs baseline: 16.2461x; 16.2461x over previous
import functools

import jax
import jax.numpy as jnp
from jax import lax
from jax.experimental import pallas as pl
from jax.experimental.pallas import tpu as pltpu
from jax.experimental.pallas import tpu_sc as plsc

N = 10000
E = 320000
NPAD = 10240
NC, NS = 2, 16
EPC = E // NC
EPT = EPC // NS
CH = 80
NCHUNK = EPT // CH
RPT = NPAD // NS

_PREC = jax.lax.Precision.HIGHEST


def _sc_deg_body(eidx, deg_out, acc, idx_v, ones_v, zb):
    c = lax.axis_index("c")
    s = lax.axis_index("s")
    for i in range(RPT // 16):
        zb[pl.ds(i * 16, 16)] = jnp.zeros((16,), jnp.float32)
    for i in range(CH // 16):
        ones_v[pl.ds(i * 16, 16)] = jnp.ones((16,), jnp.float32)
    pltpu.sync_copy(zb, acc.at[pl.ds(s * RPT, RPT)])
    plsc.subcore_barrier()
    pltpu.sync_copy(eidx.at[1, c, s], idx_v)

    def step(j, carry):
        pltpu.sync_copy(ones_v, acc.at[idx_v.at[j]], add=True)
        return carry

    lax.fori_loop(0, NCHUNK, step, 0)
    plsc.subcore_barrier()
    pltpu.sync_copy(acc.at[pl.ds(s * RPT, RPT)],
                    deg_out.at[c, pl.ds(s * RPT, RPT)])


def _sc_deg(eidx):
    mesh = plsc.VectorSubcoreMesh(core_axis_name="c", subcore_axis_name="s")
    return pl.kernel(
        _sc_deg_body,
        out_type=jax.ShapeDtypeStruct((NC, NPAD), jnp.float32),
        mesh=mesh,
        scratch_types=[
            pltpu.VMEM_SHARED((NPAD,), jnp.float32),
            pltpu.VMEM((NCHUNK, CH), jnp.int32),
            pltpu.VMEM((CH,), jnp.float32),
            pltpu.VMEM((RPT,), jnp.float32),
        ],
    )(eidx)


def _sc_conv_body(D, eidx, g, out, acc, sidx_v, didx_v, rows_v, zb, sem):
    c = lax.axis_index("c")
    s = lax.axis_index("s")
    for r in range(16):
        for q in range(D // 16):
            zb[r, pl.ds(q * 16, 16)] = jnp.zeros((16,), jnp.float32)

    def zstep(j, carry):
        pltpu.sync_copy(zb, acc.at[pl.ds(s * RPT + j * 16, 16)])
        return carry

    lax.fori_loop(0, RPT // 16, zstep, 0)
    plsc.subcore_barrier()
    pltpu.sync_copy(eidx.at[0, c, s], sidx_v)
    pltpu.sync_copy(eidx.at[1, c, s], didx_v)

    def step(j, carry):
        pltpu.async_copy(g.at[sidx_v.at[j]], rows_v, sem).wait()
        pltpu.sync_copy(rows_v, acc.at[didx_v.at[j]], add=True)
        return carry

    lax.fori_loop(0, NCHUNK, step, 0)
    plsc.subcore_barrier()
    pltpu.sync_copy(acc.at[pl.ds(s * RPT, RPT)],
                    out.at[c, pl.ds(s * RPT, RPT)])


def _sc_conv(eidx, g, D):
    mesh = plsc.VectorSubcoreMesh(core_axis_name="c", subcore_axis_name="s")
    return pl.kernel(
        functools.partial(_sc_conv_body, D),
        out_type=jax.ShapeDtypeStruct((NC, NPAD, D), jnp.float32),
        mesh=mesh,
        compiler_params=pltpu.CompilerParams(
            use_tc_tiling_on_sc=(D % 128 == 0)),
        scratch_types=[
            pltpu.VMEM_SHARED((NPAD, D), jnp.float32),
            pltpu.VMEM((NCHUNK, CH), jnp.int32),
            pltpu.VMEM((NCHUNK, CH), jnp.int32),
            pltpu.VMEM((CH, D), jnp.float32),
            pltpu.VMEM((16, D), jnp.float32),
            pltpu.SemaphoreType.DMA,
        ],
    )(eidx, g)


def _tc_wprep_body(a_ref, w_ref, o_ref):
    o_ref[...] = jnp.dot(a_ref[...], w_ref[...],
                         preferred_element_type=jnp.float32, precision=_PREC)


def _tc_wprep(a, w):
    return pl.pallas_call(
        _tc_wprep_body,
        out_shape=jax.ShapeDtypeStruct((a.shape[0], w.shape[1]), jnp.float32),
    )(a, w)


def _tc_proj_body(x_ref, w_ref, b_ref, deg_ref, g1_ref, dinv_ref):
    d = deg_ref[0, :, 0] + deg_ref[1, :, 0] + 1.0
    dinv = lax.rsqrt(d)
    hw = jnp.dot(x_ref[...], w_ref[...],
                 preferred_element_type=jnp.float32, precision=_PREC)
    hw = hw + b_ref[...]
    g1_ref[...] = hw * dinv[:, None]
    dinv_ref[...] = dinv[:, None]


def _tc_proj(x, wbig, bias1, deg, bm=400):
    nm = N // bm
    return pl.pallas_call(
        _tc_proj_body,
        grid=(nm,),
        in_specs=[
            pl.BlockSpec((bm, x.shape[1]), lambda m: (m, 0)),
            pl.BlockSpec(wbig.shape, lambda m: (0, 0)),
            pl.BlockSpec((1, bias1.shape[1]), lambda m: (0, 0)),
            pl.BlockSpec((NC, bm, 1), lambda m: (0, m, 0)),
        ],
        out_specs=[
            pl.BlockSpec((bm, wbig.shape[1]), lambda m: (m, 0)),
            pl.BlockSpec((bm, 1), lambda m: (m, 0)),
        ],
        out_shape=[
            jax.ShapeDtypeStruct((N, wbig.shape[1]), jnp.float32),
            jax.ShapeDtypeStruct((N, 1), jnp.float32),
        ],
    )(x, wbig, bias1, deg)


def _tc_mid_body(p_ref, g1_ref, dinv_ref, b1_ref, w2_ref, g2_ref):
    ssum = p_ref[0] + p_ref[1] + g1_ref[...]
    h1 = jnp.maximum(ssum * dinv_ref[...] + b1_ref[...], 0.0)
    hw2 = jnp.dot(h1, w2_ref[...],
                  preferred_element_type=jnp.float32, precision=_PREC)
    g2_ref[...] = hw2 * dinv_ref[...]


def _tc_mid(p, g1, dinv, b1, w2, bm=400):
    nm = N // bm
    hid = g1.shape[1]
    dout = w2.shape[1]
    return pl.pallas_call(
        _tc_mid_body,
        grid=(nm,),
        in_specs=[
            pl.BlockSpec((NC, bm, hid), lambda m: (0, m, 0)),
            pl.BlockSpec((bm, hid), lambda m: (m, 0)),
            pl.BlockSpec((bm, 1), lambda m: (m, 0)),
            pl.BlockSpec((1, hid), lambda m: (0, 0)),
            pl.BlockSpec((hid, dout), lambda m: (0, 0)),
        ],
        out_specs=pl.BlockSpec((bm, dout), lambda m: (m, 0)),
        out_shape=jax.ShapeDtypeStruct((N, dout), jnp.float32),
    )(p, g1, dinv, b1, w2)


def _tc_head_body(nb, q_ref, g2_ref, dinv_ref, b2_ref, m_ref, wl_ref,
                  o0_ref, o1_ref):
    w0 = wl_ref[0, 0]
    w1 = wl_ref[0, 1]
    c0 = wl_ref[0, 2]
    c1 = wl_ref[0, 3]
    for i in range(nb):
        q = q_ref[0, i] + q_ref[1, i] + g2_ref[i]
        h2 = q * dinv_ref[i] + b2_ref[...]
        t = jnp.dot(h2, m_ref[...],
                    preferred_element_type=jnp.float32, precision=_PREC)
        s = lax.dot_general(t, h2, (((1,), (1,)), ((), ())),
                            preferred_element_type=jnp.float32,
                            precision=_PREC)
        o0_ref[i] = s * w0 + c0
        o1_ref[i] = s * w1 + c1


def _tc_head(q, g2, dinv, b2, matrix, wlbl, nb=4):
    nblocks = 100
    ng = nblocks // nb
    dout = matrix.shape[0]
    return pl.pallas_call(
        functools.partial(_tc_head_body, nb),
        grid=(ng,),
        in_specs=[
            pl.BlockSpec((NC, nb, 100, dout), lambda k: (0, k, 0, 0)),
            pl.BlockSpec((nb, 100, dout), lambda k: (k, 0, 0)),
            pl.BlockSpec((nb, 100, 1), lambda k: (k, 0, 0)),
            pl.BlockSpec((1, dout), lambda k: (0, 0)),
            pl.BlockSpec((dout, dout), lambda k: (0, 0)),
            pl.BlockSpec((1, 4), lambda k: (0, 0)),
        ],
        out_specs=[
            pl.BlockSpec((nb, 100, 100), lambda k: (k, 0, 0)),
            pl.BlockSpec((nb, 100, 100), lambda k: (k, 0, 0)),
        ],
        out_shape=[
            jax.ShapeDtypeStruct((nblocks, 100, 100), jnp.float32),
            jax.ShapeDtypeStruct((nblocks, 100, 100), jnp.float32),
        ],
    )(q, g2, dinv, b2, matrix, wlbl)


def kernel(x, edge_index, W_t1, b_t1, W_t2, b_t2, W_c1, b_c1, W_c2, b_c2,
           matrix, W_lin, b_lin):
    eidx = edge_index.reshape(2, NC, NS, NCHUNK, CH)

    deg = _sc_deg(eidx)
    deg3 = deg[:, :N, None]

    a = jnp.concatenate(
        [W_t1, W_t2, (b_t1 + b_t2)[None, :], jnp.zeros((7, W_t1.shape[1]),
                                                       jnp.float32)], axis=0)
    wb = _tc_wprep(a, W_c1)
    wbig, bias1 = wb[:896], wb[896:897]

    g1, dinv = _tc_proj(x, wbig, bias1, deg3)

    p1 = _sc_conv(eidx, g1, g1.shape[1])
    g2 = _tc_mid(p1[:, :N], g1, dinv, b_c1[None, :], W_c2)

    p2 = _sc_conv(eidx, g2, g2.shape[1])

    wlbl = jnp.concatenate([W_lin, b_lin])[None, :]
    o0, o1 = _tc_head(
        p2[:, :N].reshape(NC, 100, 100, -1),
        g2.reshape(100, 100, -1),
        dinv.reshape(100, 100, 1),
        b_c2[None, :], matrix, wlbl)

    return jnp.stack([o0.reshape(-1), o1.reshape(-1)], axis=-1)


# speedup vs baseline: 25.1363x; 1.5472x over previous
import functools

import jax
import jax.numpy as jnp
from jax import lax
from jax.experimental import pallas as pl
from jax.experimental.pallas import tpu as pltpu
from jax.experimental.pallas import tpu_sc as plsc

N = 10000
E = 320000
NPAD = 10240
NC, NS = 2, 16
EPC = E // NC
EPT = EPC // NS
CH = 80
NCHUNK = EPT // CH
RPT = NPAD // NS

_PREC = jax.lax.Precision.HIGHEST


def _sc_deg_body(eidx, deg_out, acc, idx_v, ones_v, zb):
    c = lax.axis_index("c")
    s = lax.axis_index("s")
    for i in range(RPT // 16):
        zb[pl.ds(i * 16, 16)] = jnp.zeros((16,), jnp.float32)
    for i in range(CH // 16):
        ones_v[pl.ds(i * 16, 16)] = jnp.ones((16,), jnp.float32)
    pltpu.sync_copy(zb, acc.at[pl.ds(s * RPT, RPT)])
    plsc.subcore_barrier()
    pltpu.sync_copy(eidx.at[c, s], idx_v)

    def step(j, carry):
        pltpu.sync_copy(ones_v, acc.at[idx_v.at[j, 1]], add=True)
        return carry

    lax.fori_loop(0, NCHUNK, step, 0)
    plsc.subcore_barrier()
    pltpu.sync_copy(acc.at[pl.ds(s * RPT, RPT)],
                    deg_out.at[c, pl.ds(s * RPT, RPT)])


def _sc_deg(eidx):
    mesh = plsc.VectorSubcoreMesh(core_axis_name="c", subcore_axis_name="s")
    return pl.kernel(
        _sc_deg_body,
        out_type=jax.ShapeDtypeStruct((NC, NPAD), jnp.float32),
        mesh=mesh,
        scratch_types=[
            pltpu.VMEM_SHARED((NPAD,), jnp.float32),
            pltpu.VMEM((NCHUNK, 2, CH), jnp.int32),
            pltpu.VMEM((CH,), jnp.float32),
            pltpu.VMEM((RPT,), jnp.float32),
        ],
    )(eidx)


NBUF = 4


def _sc_conv_body(D, eidx, g, out, acc, idx_v, rows_v, zb, isems, gsems):
    c = lax.axis_index("c")
    s = lax.axis_index("s")
    for r in range(16):
        for q in range(D // 16):
            zb[r, pl.ds(q * 16, 16)] = jnp.zeros((16,), jnp.float32)

    def zstep(j, carry):
        pltpu.sync_copy(zb, acc.at[pl.ds(s * RPT + j * 16, 16)])
        return carry

    lax.fori_loop(0, RPT // 16, zstep, 0)
    plsc.subcore_barrier()

    def idx_issue(j, ib):
        pltpu.async_copy(eidx.at[c, s, j], idx_v.at[ib], isems.at[ib])

    def idx_wait(j, ib):
        pltpu.make_async_copy(eidx.at[c, s, j], idx_v.at[ib],
                              isems.at[ib]).wait()

    def gat_issue(ib, b):
        pltpu.async_copy(g.at[idx_v.at[ib, 0]], rows_v.at[b], gsems.at[b])

    def gat_wait(ib, b):
        pltpu.make_async_copy(g.at[idx_v.at[ib, 0]], rows_v.at[b],
                              gsems.at[b]).wait()

    for j in range(NBUF):
        idx_issue(j, j)
    for j in range(NBUF):
        idx_wait(j, j)
        gat_issue(j, j)
    for j in range(NBUF, 2 * NBUF):
        idx_issue(j, j)

    def visit(j, b, ib, ib2, refill, advance):
        gat_wait(ib, b)
        pltpu.sync_copy(rows_v.at[b], acc.at[idx_v.at[ib, 1]], add=True)
        if advance:
            idx_wait(j + NBUF, ib2)
            gat_issue(ib2, b)
        if refill:
            idx_issue(j + 2 * NBUF, ib)

    RND = 2 * NBUF
    nfull = (NCHUNK - RND) // RND

    def step(jj, carry):
        j0 = jj * RND
        for k in range(RND):
            visit(j0 + k, k % NBUF, k, (k + NBUF) % RND, True, True)
        return carry

    lax.fori_loop(0, nfull, step, 0)
    for j in range(nfull * RND, NCHUNK):
        visit(j, j % NBUF, j % RND, (j + NBUF) % RND,
              refill=(j + 2 * NBUF < NCHUNK),
              advance=(j + NBUF < NCHUNK))
    plsc.subcore_barrier()
    pltpu.sync_copy(acc.at[pl.ds(s * RPT, RPT)],
                    out.at[c, pl.ds(s * RPT, RPT)])


def _sc_conv(eidx, g, D):
    mesh = plsc.VectorSubcoreMesh(core_axis_name="c", subcore_axis_name="s")
    return pl.kernel(
        functools.partial(_sc_conv_body, D),
        out_type=jax.ShapeDtypeStruct((NC, NPAD, D), jnp.float32),
        mesh=mesh,
        compiler_params=pltpu.CompilerParams(
            use_tc_tiling_on_sc=(D % 128 == 0)),
        scratch_types=[
            pltpu.VMEM_SHARED((NPAD, D), jnp.float32),
            pltpu.VMEM((2 * NBUF, 2, CH), jnp.int32),
            pltpu.VMEM((NBUF, CH, D), jnp.float32),
            pltpu.VMEM((16, D), jnp.float32),
            pltpu.SemaphoreType.DMA((2 * NBUF,)),
            pltpu.SemaphoreType.DMA((NBUF,)),
        ],
    )(eidx, g)


def _tc_wprep_body(a_ref, w_ref, o_ref):
    o_ref[...] = jnp.dot(a_ref[...], w_ref[...],
                         preferred_element_type=jnp.float32, precision=_PREC)


def _tc_wprep(a, w):
    return pl.pallas_call(
        _tc_wprep_body,
        out_shape=jax.ShapeDtypeStruct((a.shape[0], w.shape[1]), jnp.float32),
    )(a, w)


def _tc_proj_body(x_ref, w_ref, b_ref, deg_ref, g1_ref, dinv_ref):
    d = deg_ref[0, :, 0] + deg_ref[1, :, 0] + 1.0
    dinv = lax.rsqrt(d)
    hw = jnp.dot(x_ref[...], w_ref[...],
                 preferred_element_type=jnp.float32, precision=_PREC)
    hw = hw + b_ref[...]
    g1_ref[...] = hw * dinv[:, None]
    dinv_ref[...] = dinv[:, None]


def _tc_proj(x, wbig, bias1, deg, bm=400):
    nm = N // bm
    return pl.pallas_call(
        _tc_proj_body,
        grid=(nm,),
        in_specs=[
            pl.BlockSpec((bm, x.shape[1]), lambda m: (m, 0)),
            pl.BlockSpec(wbig.shape, lambda m: (0, 0)),
            pl.BlockSpec((1, bias1.shape[1]), lambda m: (0, 0)),
            pl.BlockSpec((NC, bm, 1), lambda m: (0, m, 0)),
        ],
        out_specs=[
            pl.BlockSpec((bm, wbig.shape[1]), lambda m: (m, 0)),
            pl.BlockSpec((bm, 1), lambda m: (m, 0)),
        ],
        out_shape=[
            jax.ShapeDtypeStruct((N, wbig.shape[1]), jnp.float32),
            jax.ShapeDtypeStruct((N, 1), jnp.float32),
        ],
    )(x, wbig, bias1, deg)


def _tc_mid_body(p_ref, g1_ref, dinv_ref, b1_ref, w2_ref, g2_ref):
    ssum = p_ref[0] + p_ref[1] + g1_ref[...]
    h1 = jnp.maximum(ssum * dinv_ref[...] + b1_ref[...], 0.0)
    hw2 = jnp.dot(h1, w2_ref[...],
                  preferred_element_type=jnp.float32, precision=_PREC)
    g2_ref[...] = hw2 * dinv_ref[...]


def _tc_mid(p, g1, dinv, b1, w2, bm=400):
    nm = N // bm
    hid = g1.shape[1]
    dout = w2.shape[1]
    return pl.pallas_call(
        _tc_mid_body,
        grid=(nm,),
        in_specs=[
            pl.BlockSpec((NC, bm, hid), lambda m: (0, m, 0)),
            pl.BlockSpec((bm, hid), lambda m: (m, 0)),
            pl.BlockSpec((bm, 1), lambda m: (m, 0)),
            pl.BlockSpec((1, hid), lambda m: (0, 0)),
            pl.BlockSpec((hid, dout), lambda m: (0, 0)),
        ],
        out_specs=pl.BlockSpec((bm, dout), lambda m: (m, 0)),
        out_shape=jax.ShapeDtypeStruct((N, dout), jnp.float32),
    )(p, g1, dinv, b1, w2)


def _tc_head_body(nb, q_ref, g2_ref, dinv_ref, b2_ref, m_ref, wl_ref,
                  o0_ref, o1_ref):
    w0 = wl_ref[0, 0]
    w1 = wl_ref[0, 1]
    c0 = wl_ref[0, 2]
    c1 = wl_ref[0, 3]
    for i in range(nb):
        q = q_ref[0, i] + q_ref[1, i] + g2_ref[i]
        h2 = q * dinv_ref[i] + b2_ref[...]
        t = jnp.dot(h2, m_ref[...],
                    preferred_element_type=jnp.float32, precision=_PREC)
        s = lax.dot_general(t, h2, (((1,), (1,)), ((), ())),
                            preferred_element_type=jnp.float32,
                            precision=_PREC)
        o0_ref[i] = s * w0 + c0
        o1_ref[i] = s * w1 + c1


def _tc_head(q, g2, dinv, b2, matrix, wlbl, nb=4):
    nblocks = 100
    ng = nblocks // nb
    dout = matrix.shape[0]
    return pl.pallas_call(
        functools.partial(_tc_head_body, nb),
        grid=(ng,),
        in_specs=[
            pl.BlockSpec((NC, nb, 100, dout), lambda k: (0, k, 0, 0)),
            pl.BlockSpec((nb, 100, dout), lambda k: (k, 0, 0)),
            pl.BlockSpec((nb, 100, 1), lambda k: (k, 0, 0)),
            pl.BlockSpec((1, dout), lambda k: (0, 0)),
            pl.BlockSpec((dout, dout), lambda k: (0, 0)),
            pl.BlockSpec((1, 4), lambda k: (0, 0)),
        ],
        out_specs=[
            pl.BlockSpec((nb, 100, 100), lambda k: (k, 0, 0)),
            pl.BlockSpec((nb, 100, 100), lambda k: (k, 0, 0)),
        ],
        out_shape=[
            jax.ShapeDtypeStruct((nblocks, 100, 100), jnp.float32),
            jax.ShapeDtypeStruct((nblocks, 100, 100), jnp.float32),
        ],
    )(q, g2, dinv, b2, matrix, wlbl)


def kernel(x, edge_index, W_t1, b_t1, W_t2, b_t2, W_c1, b_c1, W_c2, b_c2,
           matrix, W_lin, b_lin):
    eidx = edge_index.reshape(2, NC, NS, NCHUNK, CH).transpose(1, 2, 3, 0, 4)

    deg = _sc_deg(eidx)
    deg3 = deg[:, :N, None]

    a = jnp.concatenate(
        [W_t1, W_t2, (b_t1 + b_t2)[None, :], jnp.zeros((7, W_t1.shape[1]),
                                                       jnp.float32)], axis=0)
    wb = _tc_wprep(a, W_c1)
    wbig, bias1 = wb[:896], wb[896:897]

    g1, dinv = _tc_proj(x, wbig, bias1, deg3)

    p1 = _sc_conv(eidx, g1, g1.shape[1])
    g2 = _tc_mid(p1[:, :N], g1, dinv, b_c1[None, :], W_c2)

    p2 = _sc_conv(eidx, g2, g2.shape[1])

    wlbl = jnp.concatenate([W_lin, b_lin])[None, :]
    o0, o1 = _tc_head(
        p2[:, :N].reshape(NC, 100, 100, -1),
        g2.reshape(100, 100, -1),
        dinv.reshape(100, 100, 1),
        b_c2[None, :], matrix, wlbl)

    return jnp.stack([o0.reshape(-1), o1.reshape(-1)], axis=-1)


# speedup vs baseline: 26.4997x; 1.0542x over previous
import functools

import jax
import jax.numpy as jnp
from jax import lax
from jax.experimental import pallas as pl
from jax.experimental.pallas import tpu as pltpu
from jax.experimental.pallas import tpu_sc as plsc

N = 10000
E = 320000
NPAD = 10240
NC, NS = 2, 16
EPC = E // NC
EPT = EPC // NS
CH = 80
NCHUNK = EPT // CH
RPT = NPAD // NS

_PREC = jax.lax.Precision.HIGHEST


DK = 5


def _sc_deg_body(eidx, deg_out, acc, idx_v, ones_v, zb, sem):
    c = lax.axis_index("c")
    s = lax.axis_index("s")
    for i in range(RPT // 16):
        zb[pl.ds(i * 16, 16)] = jnp.zeros((16,), jnp.float32)
    for i in range(CH // 16):
        ones_v[pl.ds(i * 16, 16)] = jnp.ones((16,), jnp.float32)
    pltpu.sync_copy(zb, acc.at[pl.ds(s * RPT, RPT)])
    plsc.subcore_barrier()
    pltpu.sync_copy(eidx.at[c, s], idx_v)

    def issue_round(jj):
        for k in range(DK):
            pltpu.async_copy(ones_v, acc.at[idx_v.at[jj * DK + k, 1]], sem,
                             add=True)

    def drain_round():
        for k in range(DK):
            pltpu.make_async_copy(ones_v, acc.at[idx_v.at[0, 1]], sem).wait()

    issue_round(0)

    def step(jj, carry):
        issue_round(jj)
        drain_round()
        return carry

    lax.fori_loop(1, NCHUNK // DK, step, 0)
    drain_round()
    plsc.subcore_barrier()
    pltpu.sync_copy(acc.at[pl.ds(s * RPT, RPT)],
                    deg_out.at[c, pl.ds(s * RPT, RPT)])


def _sc_deg(eidx):
    mesh = plsc.VectorSubcoreMesh(core_axis_name="c", subcore_axis_name="s")
    return pl.kernel(
        _sc_deg_body,
        out_type=jax.ShapeDtypeStruct((NC, NPAD), jnp.float32),
        mesh=mesh,
        scratch_types=[
            pltpu.VMEM_SHARED((NPAD,), jnp.float32),
            pltpu.VMEM((NCHUNK, 2, CH), jnp.int32),
            pltpu.VMEM((CH,), jnp.float32),
            pltpu.VMEM((RPT,), jnp.float32),
            pltpu.SemaphoreType.DMA,
        ],
    )(eidx)


NBUF = 4


def _sc_conv_body(D, eidx, g, out, acc, idx_v, rows_v, zb, isems, gsems):
    c = lax.axis_index("c")
    s = lax.axis_index("s")
    for r in range(16):
        for q in range(D // 16):
            zb[r, pl.ds(q * 16, 16)] = jnp.zeros((16,), jnp.float32)

    def zstep(j, carry):
        pltpu.sync_copy(zb, acc.at[pl.ds(s * RPT + j * 16, 16)])
        return carry

    lax.fori_loop(0, RPT // 16, zstep, 0)
    plsc.subcore_barrier()

    def idx_issue(j, ib):
        pltpu.async_copy(eidx.at[c, s, j], idx_v.at[ib], isems.at[ib])

    def idx_wait(j, ib):
        pltpu.make_async_copy(eidx.at[c, s, j], idx_v.at[ib],
                              isems.at[ib]).wait()

    def gat_issue(ib, b):
        pltpu.async_copy(g.at[idx_v.at[ib, 0]], rows_v.at[b], gsems.at[b])

    def gat_wait(ib, b):
        pltpu.make_async_copy(g.at[idx_v.at[ib, 0]], rows_v.at[b],
                              gsems.at[b]).wait()

    for j in range(NBUF):
        idx_issue(j, j)
    for j in range(NBUF):
        idx_wait(j, j)
        gat_issue(j, j)
    for j in range(NBUF, 2 * NBUF):
        idx_issue(j, j)

    def visit(j, b, ib, ib2, refill, advance):
        gat_wait(ib, b)
        pltpu.sync_copy(rows_v.at[b], acc.at[idx_v.at[ib, 1]], add=True)
        if advance:
            idx_wait(j + NBUF, ib2)
            gat_issue(ib2, b)
        if refill:
            idx_issue(j + 2 * NBUF, ib)

    RND = 2 * NBUF
    nfull = (NCHUNK - RND) // RND

    def step(jj, carry):
        j0 = jj * RND
        for k in range(RND):
            visit(j0 + k, k % NBUF, k, (k + NBUF) % RND, True, True)
        return carry

    lax.fori_loop(0, nfull, step, 0)
    for j in range(nfull * RND, NCHUNK):
        visit(j, j % NBUF, j % RND, (j + NBUF) % RND,
              refill=(j + 2 * NBUF < NCHUNK),
              advance=(j + NBUF < NCHUNK))
    plsc.subcore_barrier()
    pltpu.sync_copy(acc.at[pl.ds(s * RPT, RPT)],
                    out.at[c, pl.ds(s * RPT, RPT)])


def _sc_conv(eidx, g, D):
    mesh = plsc.VectorSubcoreMesh(core_axis_name="c", subcore_axis_name="s")
    return pl.kernel(
        functools.partial(_sc_conv_body, D),
        out_type=jax.ShapeDtypeStruct((NC, NPAD, D), jnp.float32),
        mesh=mesh,
        compiler_params=pltpu.CompilerParams(
            use_tc_tiling_on_sc=(D % 128 == 0)),
        scratch_types=[
            pltpu.VMEM_SHARED((NPAD, D), jnp.float32),
            pltpu.VMEM((2 * NBUF, 2, CH), jnp.int32),
            pltpu.VMEM((NBUF, CH, D), jnp.float32),
            pltpu.VMEM((16, D), jnp.float32),
            pltpu.SemaphoreType.DMA((2 * NBUF,)),
            pltpu.SemaphoreType.DMA((NBUF,)),
        ],
    )(eidx, g)


def _tc_proj_body(wt1_ref, wt2_ref, b12_ref, wc1_ref, x_ref, deg_ref,
                  g1_ref, dinv_ref, wbig_ref, bias1_ref):
    @pl.when(pl.program_id(0) == 0)
    def _():
        wc1 = wc1_ref[...]
        wbig_ref[0:128, :] = jnp.dot(wt1_ref[...], wc1,
                                     preferred_element_type=jnp.float32,
                                     precision=_PREC)
        wbig_ref[128:896, :] = jnp.dot(wt2_ref[...], wc1,
                                       preferred_element_type=jnp.float32,
                                       precision=_PREC)
        bias1_ref[...] = jnp.dot(b12_ref[...], wc1,
                                 preferred_element_type=jnp.float32,
                                 precision=_PREC)

    d = deg_ref[0, :, 0] + deg_ref[1, :, 0] + 1.0
    dinv = lax.rsqrt(d)
    hw = jnp.dot(x_ref[...], wbig_ref[...],
                 preferred_element_type=jnp.float32)
    hw = hw + bias1_ref[...]
    g1_ref[...] = hw * dinv[:, None]
    dinv_ref[...] = dinv[:, None]


def _tc_proj(wt1, wt2, b12, wc1, x, deg, bm=400):
    nm = N // bm
    hid = wc1.shape[1]
    return pl.pallas_call(
        _tc_proj_body,
        grid=(nm,),
        in_specs=[
            pl.BlockSpec(wt1.shape, lambda m: (0, 0)),
            pl.BlockSpec(wt2.shape, lambda m: (0, 0)),
            pl.BlockSpec(b12.shape, lambda m: (0, 0)),
            pl.BlockSpec(wc1.shape, lambda m: (0, 0)),
            pl.BlockSpec((bm, x.shape[1]), lambda m: (m, 0)),
            pl.BlockSpec((NC, bm, 1), lambda m: (0, m, 0)),
        ],
        out_specs=[
            pl.BlockSpec((bm, hid), lambda m: (m, 0)),
            pl.BlockSpec((bm, 1), lambda m: (m, 0)),
        ],
        out_shape=[
            jax.ShapeDtypeStruct((N, hid), jnp.float32),
            jax.ShapeDtypeStruct((N, 1), jnp.float32),
        ],
        scratch_shapes=[
            pltpu.VMEM((896, 128), jnp.float32),
            pltpu.VMEM((1, 128), jnp.float32),
        ],
    )(wt1, wt2, b12, wc1, x, deg)


def _tc_mid_body(p_ref, g1_ref, dinv_ref, b1_ref, w2_ref, g2_ref):
    ssum = p_ref[0] + p_ref[1] + g1_ref[...]
    h1 = jnp.maximum(ssum * dinv_ref[...] + b1_ref[...], 0.0)
    hw2 = jnp.dot(h1, w2_ref[...],
                  preferred_element_type=jnp.float32, precision=_PREC)
    g2_ref[...] = hw2 * dinv_ref[...]


def _tc_mid(p, g1, dinv, b1, w2, bm=400):
    nm = N // bm
    hid = g1.shape[1]
    dout = w2.shape[1]
    return pl.pallas_call(
        _tc_mid_body,
        grid=(nm,),
        in_specs=[
            pl.BlockSpec((NC, bm, hid), lambda m: (0, m, 0)),
            pl.BlockSpec((bm, hid), lambda m: (m, 0)),
            pl.BlockSpec((bm, 1), lambda m: (m, 0)),
            pl.BlockSpec((1, hid), lambda m: (0, 0)),
            pl.BlockSpec((hid, dout), lambda m: (0, 0)),
        ],
        out_specs=pl.BlockSpec((bm, dout), lambda m: (m, 0)),
        out_shape=jax.ShapeDtypeStruct((N, dout), jnp.float32),
    )(p, g1, dinv, b1, w2)


def _tc_head_body(nb, q_ref, g2_ref, dinv_ref, b2_ref, m_ref, wl_ref,
                  o0_ref, o1_ref):
    w0 = wl_ref[0, 0]
    w1 = wl_ref[0, 1]
    c0 = wl_ref[0, 2]
    c1 = wl_ref[0, 3]
    for i in range(nb):
        q = q_ref[0, i] + q_ref[1, i] + g2_ref[i]
        h2 = q * dinv_ref[i] + b2_ref[...]
        t = jnp.dot(h2, m_ref[...],
                    preferred_element_type=jnp.float32, precision=_PREC)
        s = lax.dot_general(t, h2, (((1,), (1,)), ((), ())),
                            preferred_element_type=jnp.float32,
                            precision=_PREC)
        o0_ref[i] = s * w0 + c0
        o1_ref[i] = s * w1 + c1


def _tc_head(q, g2, dinv, b2, matrix, wlbl, nb=4):
    nblocks = 100
    ng = nblocks // nb
    dout = matrix.shape[0]
    return pl.pallas_call(
        functools.partial(_tc_head_body, nb),
        grid=(ng,),
        in_specs=[
            pl.BlockSpec((NC, nb, 100, dout), lambda k: (0, k, 0, 0)),
            pl.BlockSpec((nb, 100, dout), lambda k: (k, 0, 0)),
            pl.BlockSpec((nb, 100, 1), lambda k: (k, 0, 0)),
            pl.BlockSpec((1, dout), lambda k: (0, 0)),
            pl.BlockSpec((dout, dout), lambda k: (0, 0)),
            pl.BlockSpec((1, 4), lambda k: (0, 0)),
        ],
        out_specs=[
            pl.BlockSpec((nb, 100, 100), lambda k: (k, 0, 0)),
            pl.BlockSpec((nb, 100, 100), lambda k: (k, 0, 0)),
        ],
        out_shape=[
            jax.ShapeDtypeStruct((nblocks, 100, 100), jnp.float32),
            jax.ShapeDtypeStruct((nblocks, 100, 100), jnp.float32),
        ],
    )(q, g2, dinv, b2, matrix, wlbl)


def kernel(x, edge_index, W_t1, b_t1, W_t2, b_t2, W_c1, b_c1, W_c2, b_c2,
           matrix, W_lin, b_lin):
    eidx = edge_index.reshape(2, NC, NS, NCHUNK, CH).transpose(1, 2, 3, 0, 4)

    deg = _sc_deg(eidx)
    deg3 = deg[:, :N, None]

    g1, dinv = _tc_proj(W_t1, W_t2, (b_t1 + b_t2)[None, :], W_c1, x,
                        deg3)

    p1 = _sc_conv(eidx, g1, g1.shape[1])
    g2 = _tc_mid(p1[:, :N], g1, dinv, b_c1[None, :], W_c2)

    p2 = _sc_conv(eidx, g2, g2.shape[1])

    wlbl = jnp.concatenate([W_lin, b_lin])[None, :]
    o0, o1 = _tc_head(
        p2[:, :N].reshape(NC, 100, 100, -1),
        g2.reshape(100, 100, -1),
        dinv.reshape(100, 100, 1),
        b_c2[None, :], matrix, wlbl)

    return jnp.stack([o0.reshape(-1), o1.reshape(-1)], axis=-1)


# speedup vs baseline: 30.3474x; 1.1452x over previous
import functools

import jax
import jax.numpy as jnp
from jax import lax
from jax.experimental import pallas as pl
from jax.experimental.pallas import tpu as pltpu
from jax.experimental.pallas import tpu_sc as plsc

N = 10000
E = 320000
NPAD = 10240
NC, NS = 2, 16
EPC = E // NC
EPT = EPC // NS
CH = 80
NCHUNK = EPT // CH
RPT = NPAD // NS

_PREC = jax.lax.Precision.HIGHEST


DK = 5


def _sc_deg_body(eidx, deg_out, acc, idx_v, ones_v, zb, sem):
    c = lax.axis_index("c")
    s = lax.axis_index("s")
    for i in range(RPT // 16):
        zb[pl.ds(i * 16, 16)] = jnp.zeros((16,), jnp.float32)
    for i in range(CH // 16):
        ones_v[pl.ds(i * 16, 16)] = jnp.ones((16,), jnp.float32)
    pltpu.sync_copy(zb, acc.at[pl.ds(s * RPT, RPT)])
    plsc.subcore_barrier()
    pltpu.sync_copy(eidx.at[1, c, s], idx_v)

    def issue_round(jj):
        for k in range(DK):
            pltpu.async_copy(ones_v, acc.at[idx_v.at[jj * DK + k, 0]], sem,
                             add=True)

    def drain_round():
        for k in range(DK):
            pltpu.make_async_copy(ones_v, acc.at[idx_v.at[0, 0]], sem).wait()

    issue_round(0)

    def step(jj, carry):
        issue_round(jj)
        drain_round()
        return carry

    lax.fori_loop(1, NCHUNK // DK, step, 0)
    drain_round()
    plsc.subcore_barrier()
    pltpu.sync_copy(acc.at[pl.ds(s * RPT, RPT)],
                    deg_out.at[c, pl.ds(s * RPT, RPT)])


def _sc_deg(eidx):
    mesh = plsc.VectorSubcoreMesh(core_axis_name="c", subcore_axis_name="s")
    return pl.kernel(
        _sc_deg_body,
        out_type=jax.ShapeDtypeStruct((NC, NPAD), jnp.float32),
        mesh=mesh,
        scratch_types=[
            pltpu.VMEM_SHARED((NPAD,), jnp.float32),
            pltpu.VMEM((NCHUNK, 1, CH), jnp.int32),
            pltpu.VMEM((CH,), jnp.float32),
            pltpu.VMEM((RPT,), jnp.float32),
            pltpu.SemaphoreType.DMA,
        ],
    )(eidx)


NBUF = 4


def _sc_conv_body(D, eidx, g, out, acc, sidx_v, didx_v, rows_v, zb,
                  isems, dsems, gsems):
    c = lax.axis_index("c")
    s = lax.axis_index("s")
    for r in range(16):
        for q in range(D // 16):
            zb[r, pl.ds(q * 16, 16)] = jnp.zeros((16,), jnp.float32)

    def zstep(j, carry):
        pltpu.sync_copy(zb, acc.at[pl.ds(s * RPT + j * 16, 16)])
        return carry

    lax.fori_loop(0, RPT // 16, zstep, 0)
    plsc.subcore_barrier()

    def idx_issue(j, ib):
        pltpu.async_copy(eidx.at[0, c, s, j], sidx_v.at[ib], isems.at[ib])
        pltpu.async_copy(eidx.at[1, c, s, j], didx_v.at[ib], dsems.at[ib])

    def idx_wait(j, ib):
        pltpu.make_async_copy(eidx.at[0, c, s, j], sidx_v.at[ib],
                              isems.at[ib]).wait()
        pltpu.make_async_copy(eidx.at[1, c, s, j], didx_v.at[ib],
                              dsems.at[ib]).wait()

    def gat_issue(ib, b):
        pltpu.async_copy(g.at[sidx_v.at[ib, 0]], rows_v.at[b], gsems.at[b])

    def gat_wait(ib, b):
        pltpu.make_async_copy(g.at[sidx_v.at[ib, 0]], rows_v.at[b],
                              gsems.at[b]).wait()

    for j in range(NBUF):
        idx_issue(j, j)
    for j in range(NBUF):
        idx_wait(j, j)
        gat_issue(j, j)
    for j in range(NBUF, 2 * NBUF):
        idx_issue(j, j)

    def visit(j, b, ib, ib2, refill, advance):
        gat_wait(ib, b)
        pltpu.sync_copy(rows_v.at[b], acc.at[didx_v.at[ib, 0]], add=True)
        if advance:
            idx_wait(j + NBUF, ib2)
            gat_issue(ib2, b)
        if refill:
            idx_issue(j + 2 * NBUF, ib)

    RND = 2 * NBUF
    nfull = (NCHUNK - RND) // RND

    def step(jj, carry):
        j0 = jj * RND
        for k in range(RND):
            visit(j0 + k, k % NBUF, k, (k + NBUF) % RND, True, True)
        return carry

    lax.fori_loop(0, nfull, step, 0)
    for j in range(nfull * RND, NCHUNK):
        visit(j, j % NBUF, j % RND, (j + NBUF) % RND,
              refill=(j + 2 * NBUF < NCHUNK),
              advance=(j + NBUF < NCHUNK))
    plsc.subcore_barrier()
    pltpu.sync_copy(acc.at[pl.ds(s * RPT, RPT)],
                    out.at[c, pl.ds(s * RPT, RPT)])


def _sc_conv(eidx, g, D):
    mesh = plsc.VectorSubcoreMesh(core_axis_name="c", subcore_axis_name="s")
    return pl.kernel(
        functools.partial(_sc_conv_body, D),
        out_type=jax.ShapeDtypeStruct((NC, NPAD, D), jnp.float32),
        mesh=mesh,
        compiler_params=pltpu.CompilerParams(
            use_tc_tiling_on_sc=(D % 128 == 0)),
        scratch_types=[
            pltpu.VMEM_SHARED((NPAD, D), jnp.float32),
            pltpu.VMEM((2 * NBUF, 1, CH), jnp.int32),
            pltpu.VMEM((2 * NBUF, 1, CH), jnp.int32),
            pltpu.VMEM((NBUF, CH, D), jnp.float32),
            pltpu.VMEM((16, D), jnp.float32),
            pltpu.SemaphoreType.DMA((2 * NBUF,)),
            pltpu.SemaphoreType.DMA((2 * NBUF,)),
            pltpu.SemaphoreType.DMA((NBUF,)),
        ],
    )(eidx, g)


def _tc_proj_body(wt1_ref, wt2_ref, b12_ref, wc1_ref, x_ref, deg_ref,
                  g1_ref, dinv_ref, wbig_ref, bias1_ref):
    @pl.when(pl.program_id(0) == 0)
    def _():
        wc1 = wc1_ref[...]
        wbig_ref[0:128, :] = jnp.dot(wt1_ref[...], wc1,
                                     preferred_element_type=jnp.float32,
                                     precision=_PREC)
        wbig_ref[128:896, :] = jnp.dot(wt2_ref[...], wc1,
                                       preferred_element_type=jnp.float32,
                                       precision=_PREC)
        bias1_ref[...] = jnp.dot(b12_ref[...], wc1,
                                 preferred_element_type=jnp.float32,
                                 precision=_PREC)

    d = deg_ref[0, :, 0] + deg_ref[1, :, 0] + 1.0
    dinv = lax.rsqrt(d)
    hw = jnp.dot(x_ref[...], wbig_ref[...],
                 preferred_element_type=jnp.float32)
    hw = hw + bias1_ref[...]
    g1_ref[...] = hw * dinv[:, None]
    dinv_ref[...] = dinv[:, None]


def _tc_proj(wt1, wt2, b12, wc1, x, deg, bm=400):
    nm = N // bm
    hid = wc1.shape[1]
    return pl.pallas_call(
        _tc_proj_body,
        grid=(nm,),
        in_specs=[
            pl.BlockSpec(wt1.shape, lambda m: (0, 0)),
            pl.BlockSpec(wt2.shape, lambda m: (0, 0)),
            pl.BlockSpec(b12.shape, lambda m: (0, 0)),
            pl.BlockSpec(wc1.shape, lambda m: (0, 0)),
            pl.BlockSpec((bm, x.shape[1]), lambda m: (m, 0)),
            pl.BlockSpec((NC, bm, 1), lambda m: (0, m, 0)),
        ],
        out_specs=[
            pl.BlockSpec((bm, hid), lambda m: (m, 0)),
            pl.BlockSpec((bm, 1), lambda m: (m, 0)),
        ],
        out_shape=[
            jax.ShapeDtypeStruct((N, hid), jnp.float32),
            jax.ShapeDtypeStruct((N, 1), jnp.float32),
        ],
        scratch_shapes=[
            pltpu.VMEM((896, 128), jnp.float32),
            pltpu.VMEM((1, 128), jnp.float32),
        ],
    )(wt1, wt2, b12, wc1, x, deg)


def _tc_mid_body(p_ref, g1_ref, dinv_ref, b1_ref, w2_ref, g2_ref):
    ssum = p_ref[0] + p_ref[1] + g1_ref[...]
    h1 = jnp.maximum(ssum * dinv_ref[...] + b1_ref[...], 0.0)
    hw2 = jnp.dot(h1, w2_ref[...], preferred_element_type=jnp.float32)
    g2_ref[...] = hw2 * dinv_ref[...]


def _tc_mid(p, g1, dinv, b1, w2, bm=400):
    nm = N // bm
    hid = g1.shape[1]
    dout = w2.shape[1]
    return pl.pallas_call(
        _tc_mid_body,
        grid=(nm,),
        in_specs=[
            pl.BlockSpec((NC, bm, hid), lambda m: (0, m, 0)),
            pl.BlockSpec((bm, hid), lambda m: (m, 0)),
            pl.BlockSpec((bm, 1), lambda m: (m, 0)),
            pl.BlockSpec((1, hid), lambda m: (0, 0)),
            pl.BlockSpec((hid, dout), lambda m: (0, 0)),
        ],
        out_specs=pl.BlockSpec((bm, dout), lambda m: (m, 0)),
        out_shape=jax.ShapeDtypeStruct((N, dout), jnp.float32),
    )(p, g1, dinv, b1, w2)


def _tc_head_body(nb, q_ref, g2_ref, dinv_ref, b2_ref, m_ref, wl_ref, o_ref):
    w0 = wl_ref[0, 0]
    w1 = wl_ref[0, 1]
    c0 = wl_ref[0, 2]
    c1 = wl_ref[0, 3]
    for i in range(nb):
        r = pl.ds(i * 100, 100)
        q = q_ref[0, r, :] + q_ref[1, r, :] + g2_ref[r, :]
        h2 = q * dinv_ref[r, :] + b2_ref[...]
        t = jnp.dot(h2, m_ref[...],
                    preferred_element_type=jnp.float32, precision=_PREC)
        s = lax.dot_general(t, h2, (((1,), (1,)), ((), ())),
                            preferred_element_type=jnp.float32,
                            precision=_PREC)
        o_ref[0, i] = s * w0 + c0
        o_ref[1, i] = s * w1 + c1


def _tc_head(q, g2, dinv, b2, matrix, wlbl, nb=4):
    bm = nb * 100
    ng = 100 // nb
    dout = matrix.shape[1]
    return pl.pallas_call(
        functools.partial(_tc_head_body, nb),
        grid=(ng,),
        in_specs=[
            pl.BlockSpec((NC, bm, dout), lambda k: (0, k, 0)),
            pl.BlockSpec((bm, dout), lambda k: (k, 0)),
            pl.BlockSpec((bm, 1), lambda k: (k, 0)),
            pl.BlockSpec((1, dout), lambda k: (0, 0)),
            pl.BlockSpec((dout, dout), lambda k: (0, 0)),
            pl.BlockSpec((1, 4), lambda k: (0, 0)),
        ],
        out_specs=pl.BlockSpec((2, nb, 100, 100), lambda k: (0, k, 0, 0)),
        out_shape=jax.ShapeDtypeStruct((2, 100, 100, 100), jnp.float32),
    )(q, g2, dinv, b2, matrix, wlbl)


def kernel(x, edge_index, W_t1, b_t1, W_t2, b_t2, W_c1, b_c1, W_c2, b_c2,
           matrix, W_lin, b_lin):
    eidx = edge_index.reshape(2, NC, NS, NCHUNK, 1, CH)

    deg = _sc_deg(eidx)
    deg3 = deg[:, :N, None]

    g1, dinv = _tc_proj(W_t1, W_t2, (b_t1 + b_t2)[None, :], W_c1, x,
                        deg3)

    p1 = _sc_conv(eidx, g1, g1.shape[1])
    g2 = _tc_mid(p1, g1, dinv, b_c1[None, :], W_c2)

    p2 = _sc_conv(eidx, g2, g2.shape[1])

    wlbl = jnp.concatenate([W_lin, b_lin])[None, :]
    o = _tc_head(p2, g2, dinv, b_c2[None, :], matrix, wlbl)

    return o.reshape(2, -1).T


# speedup vs baseline: 31.0969x; 1.0247x over previous
import functools

import jax
import jax.numpy as jnp
from jax import lax
from jax.experimental import pallas as pl
from jax.experimental.pallas import tpu as pltpu
from jax.experimental.pallas import tpu_sc as plsc

N = 10000
E = 320000
NPAD = 10240
NC, NS = 2, 16
NW = NC * NS
CH = 128
TOTCH = E // CH
NCHUNK = TOTCH // NW
NEXTRA = TOTCH - NCHUNK * NW
RPT = NPAD // NS

_PREC = jax.lax.Precision.HIGHEST


NBUF = 4


def _sc_deg_body(eidx, deg_out, acc, idx_v, ones_v, zb, isems, ssems):
    c = lax.axis_index("c")
    s = lax.axis_index("s")
    w = c * NS + s
    for i in range(RPT // 16):
        zb[pl.ds(i * 16, 16)] = jnp.zeros((16,), jnp.float32)
    for i in range(CH // 16):
        ones_v[pl.ds(i * 16, 16)] = jnp.ones((16,), jnp.float32)
    pltpu.sync_copy(zb, acc.at[pl.ds(s * RPT, RPT)])
    plsc.subcore_barrier()

    def idx_issue(j, ib):
        pltpu.async_copy(eidx.at[:, pl.ds((j * NW + w) * CH, CH)],
                         idx_v.at[ib], isems.at[ib])

    def idx_wait(j, ib):
        pltpu.make_async_copy(eidx.at[:, pl.ds((j * NW + w) * CH, CH)],
                              idx_v.at[ib], isems.at[ib]).wait()

    def visit(j, b, ib, ibp, drain, refill):
        idx_wait(j, ib)
        if drain:
            pltpu.make_async_copy(ones_v, acc.at[idx_v.at[ib, 1]],
                                  ssems.at[b]).wait()
        pltpu.async_copy(ones_v, acc.at[idx_v.at[ib, 1]], ssems.at[b],
                         add=True)
        if refill:
            idx_issue(j + NBUF, ibp)

    for j in range(NBUF):
        idx_issue(j, j)
    RND = 2 * NBUF
    for j in range(NBUF):
        visit(j, j % NBUF, j % RND, (j + NBUF) % RND,
              drain=False, refill=True)

    nfull = (NCHUNK - 2 * NBUF) // RND

    def stepw(jj, carry):
        j0 = NBUF + jj * RND
        for k in range(RND):
            j = j0 + k
            visit(j, (NBUF + k) % NBUF, (NBUF + k) % RND, k % RND,
                  drain=True, refill=True)
        return carry

    lax.fori_loop(0, nfull, stepw, 0)
    for j in range(NBUF + nfull * RND, NCHUNK):
        visit(j, j % NBUF, j % RND, (j + NBUF) % RND,
              drain=True, refill=(j + NBUF < NCHUNK))
    for b in range(NBUF):
        pltpu.make_async_copy(ones_v, acc.at[idx_v.at[0, 1]],
                              ssems.at[b]).wait()

    @pl.when(w < NEXTRA)
    def _():
        pltpu.sync_copy(eidx.at[:, pl.ds((NCHUNK * NW + w) * CH, CH)],
                        idx_v.at[0])
        pltpu.sync_copy(ones_v, acc.at[idx_v.at[0, 1]], add=True)

    plsc.subcore_barrier()
    pltpu.sync_copy(acc.at[pl.ds(s * RPT, RPT)],
                    deg_out.at[c, pl.ds(s * RPT, RPT)])


def _sc_deg(eidx):
    mesh = plsc.VectorSubcoreMesh(core_axis_name="c", subcore_axis_name="s")
    return pl.kernel(
        _sc_deg_body,
        out_type=jax.ShapeDtypeStruct((NC, NPAD), jnp.float32),
        mesh=mesh,
        scratch_types=[
            pltpu.VMEM_SHARED((NPAD,), jnp.float32),
            pltpu.VMEM((2 * NBUF, 2, CH), jnp.int32),
            pltpu.VMEM((CH,), jnp.float32),
            pltpu.VMEM((RPT,), jnp.float32),
            pltpu.SemaphoreType.DMA((2 * NBUF,)),
            pltpu.SemaphoreType.DMA((NBUF,)),
        ],
    )(eidx)


def _sc_conv_body(D, NB, eidx, g, out, acc, idx_v, rows_v, zb, isems, gsems):
    c = lax.axis_index("c")
    s = lax.axis_index("s")
    w = c * NS + s
    for r in range(16):
        for q in range(D // 16):
            zb[r, pl.ds(q * 16, 16)] = jnp.zeros((16,), jnp.float32)

    def zstep(j, carry):
        pltpu.sync_copy(zb, acc.at[pl.ds(s * RPT + j * 16, 16)])
        return carry

    lax.fori_loop(0, RPT // 16, zstep, 0)
    plsc.subcore_barrier()

    def idx_issue(j, ib):
        pltpu.async_copy(eidx.at[:, pl.ds((j * NW + w) * CH, CH)],
                         idx_v.at[ib], isems.at[ib])

    def idx_wait(j, ib):
        pltpu.make_async_copy(eidx.at[:, pl.ds((j * NW + w) * CH, CH)],
                              idx_v.at[ib], isems.at[ib]).wait()

    def gat_issue(ib, b):
        pltpu.async_copy(g.at[idx_v.at[ib, 0]], rows_v.at[b], gsems.at[b])

    def gat_wait(ib, b):
        pltpu.make_async_copy(g.at[idx_v.at[ib, 0]], rows_v.at[b],
                              gsems.at[b]).wait()

    for j in range(NB):
        idx_issue(j, j)
    for j in range(NB):
        idx_wait(j, j)
        gat_issue(j, j)
    for j in range(NB, 2 * NB):
        idx_issue(j, j)

    def visit(j, b, ib, ib2, refill, advance):
        gat_wait(ib, b)
        pltpu.sync_copy(rows_v.at[b], acc.at[idx_v.at[ib, 1]], add=True)
        if advance:
            idx_wait(j + NB, ib2)
            gat_issue(ib2, b)
        if refill:
            idx_issue(j + 2 * NB, ib)

    RND = 2 * NB
    nfull = (NCHUNK - RND) // RND

    def step(jj, carry):
        j0 = jj * RND
        for k in range(RND):
            visit(j0 + k, k % NB, k, (k + NB) % RND, True, True)
        return carry

    lax.fori_loop(0, nfull, step, 0)
    for j in range(nfull * RND, NCHUNK):
        visit(j, j % NB, j % RND, (j + NB) % RND,
              refill=(j + 2 * NB < NCHUNK),
              advance=(j + NB < NCHUNK))

    @pl.when(w < NEXTRA)
    def _():
        pltpu.sync_copy(eidx.at[:, pl.ds((NCHUNK * NW + w) * CH, CH)],
                        idx_v.at[0])
        pltpu.async_copy(g.at[idx_v.at[0, 0]], rows_v.at[0],
                         gsems.at[0]).wait()
        pltpu.sync_copy(rows_v.at[0], acc.at[idx_v.at[0, 1]], add=True)

    plsc.subcore_barrier()
    pltpu.sync_copy(acc.at[pl.ds(s * RPT, RPT)],
                    out.at[c, pl.ds(s * RPT, RPT)])


def _sc_conv(eidx, g, D):
    NB = 2 if D == 128 else 4
    mesh = plsc.VectorSubcoreMesh(core_axis_name="c", subcore_axis_name="s")
    return pl.kernel(
        functools.partial(_sc_conv_body, D, NB),
        out_type=jax.ShapeDtypeStruct((NC, NPAD, D), jnp.float32),
        mesh=mesh,
        compiler_params=pltpu.CompilerParams(
            use_tc_tiling_on_sc=(D % 128 == 0)),
        scratch_types=[
            pltpu.VMEM_SHARED((NPAD, D), jnp.float32),
            pltpu.VMEM((2 * NB, 2, CH), jnp.int32),
            pltpu.VMEM((NB, CH, D), jnp.float32),
            pltpu.VMEM((16, D), jnp.float32),
            pltpu.SemaphoreType.DMA((2 * NB,)),
            pltpu.SemaphoreType.DMA((NB,)),
        ],
    )(eidx, g)


def _tc_proj_body(wt1_ref, wt2_ref, b12_ref, wc1_ref, x_ref, deg_ref,
                  g1_ref, dinv_ref, wbig_ref, bias1_ref):
    @pl.when(pl.program_id(0) == 0)
    def _():
        wc1 = wc1_ref[...]
        wbig_ref[0:128, :] = jnp.dot(wt1_ref[...], wc1,
                                     preferred_element_type=jnp.float32,
                                     precision=_PREC)
        wbig_ref[128:896, :] = jnp.dot(wt2_ref[...], wc1,
                                       preferred_element_type=jnp.float32,
                                       precision=_PREC)
        bias1_ref[...] = jnp.dot(b12_ref[...], wc1,
                                 preferred_element_type=jnp.float32,
                                 precision=_PREC)

    d = deg_ref[0, :, 0] + deg_ref[1, :, 0] + 1.0
    dinv = lax.rsqrt(d)
    hw = jnp.dot(x_ref[...], wbig_ref[...],
                 preferred_element_type=jnp.float32)
    hw = hw + bias1_ref[...]
    g1_ref[...] = hw * dinv[:, None]
    dinv_ref[...] = dinv[:, None]


def _tc_proj(wt1, wt2, b12, wc1, x, deg, bm=1000):
    nm = N // bm
    hid = wc1.shape[1]
    return pl.pallas_call(
        _tc_proj_body,
        grid=(nm,),
        in_specs=[
            pl.BlockSpec(wt1.shape, lambda m: (0, 0)),
            pl.BlockSpec(wt2.shape, lambda m: (0, 0)),
            pl.BlockSpec(b12.shape, lambda m: (0, 0)),
            pl.BlockSpec(wc1.shape, lambda m: (0, 0)),
            pl.BlockSpec((bm, x.shape[1]), lambda m: (m, 0)),
            pl.BlockSpec((NC, bm, 1), lambda m: (0, m, 0)),
        ],
        out_specs=[
            pl.BlockSpec((bm, hid), lambda m: (m, 0)),
            pl.BlockSpec((bm, 1), lambda m: (m, 0)),
        ],
        out_shape=[
            jax.ShapeDtypeStruct((N, hid), jnp.float32),
            jax.ShapeDtypeStruct((N, 1), jnp.float32),
        ],
        scratch_shapes=[
            pltpu.VMEM((896, 128), jnp.float32),
            pltpu.VMEM((1, 128), jnp.float32),
        ],
    )(wt1, wt2, b12, wc1, x, deg)


def _tc_mid_body(p_ref, g1_ref, dinv_ref, b1_ref, w2_ref, g2_ref):
    ssum = p_ref[0] + p_ref[1] + g1_ref[...]
    h1 = jnp.maximum(ssum * dinv_ref[...] + b1_ref[...], 0.0)
    hw2 = jnp.dot(h1, w2_ref[...], preferred_element_type=jnp.float32)
    g2_ref[...] = hw2 * dinv_ref[...]


def _tc_mid(p, g1, dinv, b1, w2, bm=1000):
    nm = N // bm
    hid = g1.shape[1]
    dout = w2.shape[1]
    return pl.pallas_call(
        _tc_mid_body,
        grid=(nm,),
        in_specs=[
            pl.BlockSpec((NC, bm, hid), lambda m: (0, m, 0)),
            pl.BlockSpec((bm, hid), lambda m: (m, 0)),
            pl.BlockSpec((bm, 1), lambda m: (m, 0)),
            pl.BlockSpec((1, hid), lambda m: (0, 0)),
            pl.BlockSpec((hid, dout), lambda m: (0, 0)),
        ],
        out_specs=pl.BlockSpec((bm, dout), lambda m: (m, 0)),
        out_shape=jax.ShapeDtypeStruct((N, dout), jnp.float32),
    )(p, g1, dinv, b1, w2)


def _tc_head_body(nb, q_ref, g2_ref, dinv_ref, b2_ref, m_ref, wl_ref, o_ref):
    w0 = wl_ref[0, 0]
    w1 = wl_ref[0, 1]
    c0 = wl_ref[0, 2]
    c1 = wl_ref[0, 3]
    for i in range(nb):
        r = pl.ds(i * 100, 100)
        q = q_ref[0, r, :] + q_ref[1, r, :] + g2_ref[r, :]
        h2 = q * dinv_ref[r, :] + b2_ref[...]
        t = jnp.dot(h2, m_ref[...],
                    preferred_element_type=jnp.float32, precision=_PREC)
        s = lax.dot_general(t, h2, (((1,), (1,)), ((), ())),
                            preferred_element_type=jnp.float32,
                            precision=_PREC)
        o_ref[0, i] = s * w0 + c0
        o_ref[1, i] = s * w1 + c1


def _tc_head(q, g2, dinv, b2, matrix, wlbl, nb=4):
    bm = nb * 100
    ng = 100 // nb
    dout = matrix.shape[1]
    return pl.pallas_call(
        functools.partial(_tc_head_body, nb),
        grid=(ng,),
        in_specs=[
            pl.BlockSpec((NC, bm, dout), lambda k: (0, k, 0)),
            pl.BlockSpec((bm, dout), lambda k: (k, 0)),
            pl.BlockSpec((bm, 1), lambda k: (k, 0)),
            pl.BlockSpec((1, dout), lambda k: (0, 0)),
            pl.BlockSpec((dout, dout), lambda k: (0, 0)),
            pl.BlockSpec((1, 4), lambda k: (0, 0)),
        ],
        out_specs=pl.BlockSpec((2, nb, 100, 100), lambda k: (0, k, 0, 0)),
        out_shape=jax.ShapeDtypeStruct((2, 100, 100, 100), jnp.float32),
    )(q, g2, dinv, b2, matrix, wlbl)


def kernel(x, edge_index, W_t1, b_t1, W_t2, b_t2, W_c1, b_c1, W_c2, b_c2,
           matrix, W_lin, b_lin):
    eidx = edge_index

    deg = _sc_deg(eidx)
    deg3 = deg[:, :N, None]

    g1, dinv = _tc_proj(W_t1, W_t2, (b_t1 + b_t2)[None, :], W_c1, x,
                        deg3)

    p1 = _sc_conv(eidx, g1, g1.shape[1])
    g2 = _tc_mid(p1, g1, dinv, b_c1[None, :], W_c2)

    p2 = _sc_conv(eidx, g2, g2.shape[1])

    wlbl = jnp.concatenate([W_lin, b_lin])[None, :]
    o = _tc_head(p2, g2, dinv, b_c2[None, :], matrix, wlbl)

    return o.reshape(2, -1).T


# speedup vs baseline: 32.6693x; 1.0506x over previous
import functools

import jax
import jax.numpy as jnp
from jax import lax
from jax.experimental import pallas as pl
from jax.experimental.pallas import tpu as pltpu
from jax.experimental.pallas import tpu_sc as plsc

N = 10000
E = 320000
NPAD = 10240
NC, NS = 2, 16
NW = NC * NS
CH = 128
TOTCH = E // CH
NCHUNK = TOTCH // NW
NEXTRA = TOTCH - NCHUNK * NW
RPT = NPAD // NS

_PREC = jax.lax.Precision.HIGHEST


NBUF = 4


def _sc_deg_body(eidx, deg_out, acc, idx_v, ones_v, zb, isems, ssems):
    c = lax.axis_index("c")
    s = lax.axis_index("s")
    w = c * NS + s
    for i in range(RPT // 16):
        zb[pl.ds(i * 16, 16)] = jnp.zeros((16,), jnp.float32)
    for i in range(CH // 16):
        ones_v[pl.ds(i * 16, 16)] = jnp.ones((16,), jnp.float32)
    pltpu.sync_copy(zb, acc.at[pl.ds(s * RPT, RPT)])
    plsc.subcore_barrier()

    def idx_issue(j, ib):
        pltpu.async_copy(eidx.at[:, pl.ds((j * NW + w) * CH, CH)],
                         idx_v.at[ib], isems.at[ib])

    def idx_wait(j, ib):
        pltpu.make_async_copy(eidx.at[:, pl.ds((j * NW + w) * CH, CH)],
                              idx_v.at[ib], isems.at[ib]).wait()

    def visit(j, b, ib, ibp, drain, refill):
        idx_wait(j, ib)
        if drain:
            pltpu.make_async_copy(ones_v, acc.at[idx_v.at[ib, 1]],
                                  ssems.at[b]).wait()
        pltpu.async_copy(ones_v, acc.at[idx_v.at[ib, 1]], ssems.at[b],
                         add=True)
        if refill:
            idx_issue(j + NBUF, ibp)

    for j in range(NBUF):
        idx_issue(j, j)
    RND = 2 * NBUF
    for j in range(NBUF):
        visit(j, j % NBUF, j % RND, (j + NBUF) % RND,
              drain=False, refill=True)

    nfull = (NCHUNK - 2 * NBUF) // RND

    def stepw(jj, carry):
        j0 = NBUF + jj * RND
        for k in range(RND):
            j = j0 + k
            visit(j, (NBUF + k) % NBUF, (NBUF + k) % RND, k % RND,
                  drain=True, refill=True)
        return carry

    lax.fori_loop(0, nfull, stepw, 0)
    for j in range(NBUF + nfull * RND, NCHUNK):
        visit(j, j % NBUF, j % RND, (j + NBUF) % RND,
              drain=True, refill=(j + NBUF < NCHUNK))
    for b in range(NBUF):
        pltpu.make_async_copy(ones_v, acc.at[idx_v.at[0, 1]],
                              ssems.at[b]).wait()

    @pl.when(w < NEXTRA)
    def _():
        pltpu.sync_copy(eidx.at[:, pl.ds((NCHUNK * NW + w) * CH, CH)],
                        idx_v.at[0])
        pltpu.sync_copy(ones_v, acc.at[idx_v.at[0, 1]], add=True)

    plsc.subcore_barrier()
    pltpu.sync_copy(acc.at[pl.ds(s * RPT, RPT)],
                    deg_out.at[c, pl.ds(s * RPT, RPT)])


def _sc_deg(eidx):
    mesh = plsc.VectorSubcoreMesh(core_axis_name="c", subcore_axis_name="s")
    return pl.kernel(
        _sc_deg_body,
        out_type=jax.ShapeDtypeStruct((NC, NPAD), jnp.float32),
        mesh=mesh,
        scratch_types=[
            pltpu.VMEM_SHARED((NPAD,), jnp.float32),
            pltpu.VMEM((2 * NBUF, 2, CH), jnp.int32),
            pltpu.VMEM((CH,), jnp.float32),
            pltpu.VMEM((RPT,), jnp.float32),
            pltpu.SemaphoreType.DMA((2 * NBUF,)),
            pltpu.SemaphoreType.DMA((NBUF,)),
        ],
    )(eidx)


CH1 = 80
NCHUNK1 = (E // NW) // CH1


def _sc_conv1_body(D, NB, eidx, g, out, acc, sidx_v, didx_v, rows_v, zb,
                   isems, dsems, gsems):
    c = lax.axis_index("c")
    s = lax.axis_index("s")
    for r in range(16):
        for q in range(D // 16):
            zb[r, pl.ds(q * 16, 16)] = jnp.zeros((16,), jnp.float32)

    def zstep(j, carry):
        pltpu.sync_copy(zb, acc.at[pl.ds(s * RPT + j * 16, 16)])
        return carry

    lax.fori_loop(0, RPT // 16, zstep, 0)
    plsc.subcore_barrier()

    def idx_issue(j, ib):
        pltpu.async_copy(eidx.at[0, c, s, j], sidx_v.at[ib], isems.at[ib])
        pltpu.async_copy(eidx.at[1, c, s, j], didx_v.at[ib], dsems.at[ib])

    def idx_wait(j, ib):
        pltpu.make_async_copy(eidx.at[0, c, s, j], sidx_v.at[ib],
                              isems.at[ib]).wait()
        pltpu.make_async_copy(eidx.at[1, c, s, j], didx_v.at[ib],
                              dsems.at[ib]).wait()

    def gat_issue(ib, b):
        pltpu.async_copy(g.at[sidx_v.at[ib, 0]], rows_v.at[b], gsems.at[b])

    def gat_wait(ib, b):
        pltpu.make_async_copy(g.at[sidx_v.at[ib, 0]], rows_v.at[b],
                              gsems.at[b]).wait()

    for j in range(NB):
        idx_issue(j, j)
    for j in range(NB):
        idx_wait(j, j)
        gat_issue(j, j)
    for j in range(NB, 2 * NB):
        idx_issue(j, j)

    def visit(j, b, ib, ib2, refill, advance):
        gat_wait(ib, b)
        pltpu.sync_copy(rows_v.at[b], acc.at[didx_v.at[ib, 0]], add=True)
        if advance:
            idx_wait(j + NB, ib2)
            gat_issue(ib2, b)
        if refill:
            idx_issue(j + 2 * NB, ib)

    RND = 2 * NB
    nfull = (NCHUNK1 - RND) // RND

    def step(jj, carry):
        j0 = jj * RND
        for k in range(RND):
            visit(j0 + k, k % NB, k, (k + NB) % RND, True, True)
        return carry

    lax.fori_loop(0, nfull, step, 0)
    for j in range(nfull * RND, NCHUNK1):
        visit(j, j % NB, j % RND, (j + NB) % RND,
              refill=(j + 2 * NB < NCHUNK1),
              advance=(j + NB < NCHUNK1))
    plsc.subcore_barrier()
    pltpu.sync_copy(acc.at[pl.ds(s * RPT, RPT)],
                    out.at[c, pl.ds(s * RPT, RPT)])


def _sc_conv1(eidx6, g):
    D, NB = 128, 4
    mesh = plsc.VectorSubcoreMesh(core_axis_name="c", subcore_axis_name="s")
    return pl.kernel(
        functools.partial(_sc_conv1_body, D, NB),
        out_type=jax.ShapeDtypeStruct((NC, NPAD, D), jnp.float32),
        mesh=mesh,
        compiler_params=pltpu.CompilerParams(use_tc_tiling_on_sc=True),
        scratch_types=[
            pltpu.VMEM_SHARED((NPAD, D), jnp.float32),
            pltpu.VMEM((2 * NB, 1, CH1), jnp.int32),
            pltpu.VMEM((2 * NB, 1, CH1), jnp.int32),
            pltpu.VMEM((NB, CH1, D), jnp.float32),
            pltpu.VMEM((16, D), jnp.float32),
            pltpu.SemaphoreType.DMA((2 * NB,)),
            pltpu.SemaphoreType.DMA((2 * NB,)),
            pltpu.SemaphoreType.DMA((NB,)),
        ],
    )(eidx6, g)


def _sc_conv_body(D, NB, eidx, g, out, acc, idx_v, rows_v, zb, isems, gsems):
    c = lax.axis_index("c")
    s = lax.axis_index("s")
    w = c * NS + s
    for r in range(16):
        for q in range(D // 16):
            zb[r, pl.ds(q * 16, 16)] = jnp.zeros((16,), jnp.float32)

    def zstep(j, carry):
        pltpu.sync_copy(zb, acc.at[pl.ds(s * RPT + j * 16, 16)])
        return carry

    lax.fori_loop(0, RPT // 16, zstep, 0)
    plsc.subcore_barrier()

    def idx_issue(j, ib):
        pltpu.async_copy(eidx.at[:, pl.ds((j * NW + w) * CH, CH)],
                         idx_v.at[ib], isems.at[ib])

    def idx_wait(j, ib):
        pltpu.make_async_copy(eidx.at[:, pl.ds((j * NW + w) * CH, CH)],
                              idx_v.at[ib], isems.at[ib]).wait()

    def gat_issue(ib, b):
        pltpu.async_copy(g.at[idx_v.at[ib, 0]], rows_v.at[b], gsems.at[b])

    def gat_wait(ib, b):
        pltpu.make_async_copy(g.at[idx_v.at[ib, 0]], rows_v.at[b],
                              gsems.at[b]).wait()

    for j in range(NB):
        idx_issue(j, j)
    for j in range(NB):
        idx_wait(j, j)
        gat_issue(j, j)
    for j in range(NB, 2 * NB):
        idx_issue(j, j)

    def visit(j, b, ib, ib2, refill, advance):
        gat_wait(ib, b)
        pltpu.sync_copy(rows_v.at[b], acc.at[idx_v.at[ib, 1]], add=True)
        if advance:
            idx_wait(j + NB, ib2)
            gat_issue(ib2, b)
        if refill:
            idx_issue(j + 2 * NB, ib)

    RND = 2 * NB
    nfull = (NCHUNK - RND) // RND

    def step(jj, carry):
        j0 = jj * RND
        for k in range(RND):
            visit(j0 + k, k % NB, k, (k + NB) % RND, True, True)
        return carry

    lax.fori_loop(0, nfull, step, 0)
    for j in range(nfull * RND, NCHUNK):
        visit(j, j % NB, j % RND, (j + NB) % RND,
              refill=(j + 2 * NB < NCHUNK),
              advance=(j + NB < NCHUNK))

    @pl.when(w < NEXTRA)
    def _():
        pltpu.sync_copy(eidx.at[:, pl.ds((NCHUNK * NW + w) * CH, CH)],
                        idx_v.at[0])
        pltpu.async_copy(g.at[idx_v.at[0, 0]], rows_v.at[0],
                         gsems.at[0]).wait()
        pltpu.sync_copy(rows_v.at[0], acc.at[idx_v.at[0, 1]], add=True)

    plsc.subcore_barrier()
    pltpu.sync_copy(acc.at[pl.ds(s * RPT, RPT)],
                    out.at[c, pl.ds(s * RPT, RPT)])


def _sc_conv(eidx, g, D):
    NB = 2 if D == 128 else 4
    mesh = plsc.VectorSubcoreMesh(core_axis_name="c", subcore_axis_name="s")
    return pl.kernel(
        functools.partial(_sc_conv_body, D, NB),
        out_type=jax.ShapeDtypeStruct((NC, NPAD, D), jnp.float32),
        mesh=mesh,
        compiler_params=pltpu.CompilerParams(
            use_tc_tiling_on_sc=(D % 128 == 0)),
        scratch_types=[
            pltpu.VMEM_SHARED((NPAD, D), jnp.float32),
            pltpu.VMEM((2 * NB, 2, CH), jnp.int32),
            pltpu.VMEM((NB, CH, D), jnp.float32),
            pltpu.VMEM((16, D), jnp.float32),
            pltpu.SemaphoreType.DMA((2 * NB,)),
            pltpu.SemaphoreType.DMA((NB,)),
        ],
    )(eidx, g)


def _tc_proj_body(wt1_ref, wt2_ref, b12_ref, wc1_ref, x_ref, deg_ref,
                  g1_ref, dinv_ref, wbig_ref, bias1_ref):
    @pl.when(pl.program_id(0) == 0)
    def _():
        wc1 = wc1_ref[...]
        wbig_ref[0:128, :] = jnp.dot(wt1_ref[...], wc1,
                                     preferred_element_type=jnp.float32,
                                     precision=_PREC)
        wbig_ref[128:896, :] = jnp.dot(wt2_ref[...], wc1,
                                       preferred_element_type=jnp.float32,
                                       precision=_PREC)
        bias1_ref[...] = jnp.dot(b12_ref[...], wc1,
                                 preferred_element_type=jnp.float32,
                                 precision=_PREC)

    d = deg_ref[0, :, 0] + deg_ref[1, :, 0] + 1.0
    dinv = lax.rsqrt(d)
    hw = jnp.dot(x_ref[...], wbig_ref[...],
                 preferred_element_type=jnp.float32)
    hw = hw + bias1_ref[...]
    g1_ref[...] = hw * dinv[:, None]
    dinv_ref[...] = dinv[:, None]


def _tc_proj(wt1, wt2, b12, wc1, x, deg, bm=1000):
    nm = N // bm
    hid = wc1.shape[1]
    return pl.pallas_call(
        _tc_proj_body,
        grid=(nm,),
        in_specs=[
            pl.BlockSpec(wt1.shape, lambda m: (0, 0)),
            pl.BlockSpec(wt2.shape, lambda m: (0, 0)),
            pl.BlockSpec(b12.shape, lambda m: (0, 0)),
            pl.BlockSpec(wc1.shape, lambda m: (0, 0)),
            pl.BlockSpec((bm, x.shape[1]), lambda m: (m, 0)),
            pl.BlockSpec((NC, bm, 1), lambda m: (0, m, 0)),
        ],
        out_specs=[
            pl.BlockSpec((bm, hid), lambda m: (m, 0)),
            pl.BlockSpec((bm, 1), lambda m: (m, 0)),
        ],
        out_shape=[
            jax.ShapeDtypeStruct((N, hid), jnp.float32),
            jax.ShapeDtypeStruct((N, 1), jnp.float32),
        ],
        scratch_shapes=[
            pltpu.VMEM((896, 128), jnp.float32),
            pltpu.VMEM((1, 128), jnp.float32),
        ],
    )(wt1, wt2, b12, wc1, x, deg)


def _tc_mid_body(p_ref, g1_ref, dinv_ref, b1_ref, w2_ref, g2_ref):
    ssum = p_ref[0] + p_ref[1] + g1_ref[...]
    h1 = jnp.maximum(ssum * dinv_ref[...] + b1_ref[...], 0.0)
    hw2 = jnp.dot(h1, w2_ref[...], preferred_element_type=jnp.float32)
    g2_ref[...] = hw2 * dinv_ref[...]


def _tc_mid(p, g1, dinv, b1, w2, bm=1000):
    nm = N // bm
    hid = g1.shape[1]
    dout = w2.shape[1]
    return pl.pallas_call(
        _tc_mid_body,
        grid=(nm,),
        in_specs=[
            pl.BlockSpec((NC, bm, hid), lambda m: (0, m, 0)),
            pl.BlockSpec((bm, hid), lambda m: (m, 0)),
            pl.BlockSpec((bm, 1), lambda m: (m, 0)),
            pl.BlockSpec((1, hid), lambda m: (0, 0)),
            pl.BlockSpec((hid, dout), lambda m: (0, 0)),
        ],
        out_specs=pl.BlockSpec((bm, dout), lambda m: (m, 0)),
        out_shape=jax.ShapeDtypeStruct((N, dout), jnp.float32),
    )(p, g1, dinv, b1, w2)


def _tc_head_body(nb, q_ref, g2_ref, dinv_ref, b2_ref, m_ref, wl_ref, o_ref):
    w0 = wl_ref[0, 0]
    w1 = wl_ref[0, 1]
    c0 = wl_ref[0, 2]
    c1 = wl_ref[0, 3]
    for i in range(nb):
        r = pl.ds(i * 100, 100)
        q = q_ref[0, r, :] + q_ref[1, r, :] + g2_ref[r, :]
        h2 = q * dinv_ref[r, :] + b2_ref[...]
        t = jnp.dot(h2, m_ref[...],
                    preferred_element_type=jnp.float32, precision=_PREC)
        s = lax.dot_general(t, h2, (((1,), (1,)), ((), ())),
                            preferred_element_type=jnp.float32,
                            precision=_PREC)
        o_ref[0, i] = s * w0 + c0
        o_ref[1, i] = s * w1 + c1


def _tc_head(q, g2, dinv, b2, matrix, wlbl, nb=4):
    bm = nb * 100
    ng = 100 // nb
    dout = matrix.shape[1]
    return pl.pallas_call(
        functools.partial(_tc_head_body, nb),
        grid=(ng,),
        in_specs=[
            pl.BlockSpec((NC, bm, dout), lambda k: (0, k, 0)),
            pl.BlockSpec((bm, dout), lambda k: (k, 0)),
            pl.BlockSpec((bm, 1), lambda k: (k, 0)),
            pl.BlockSpec((1, dout), lambda k: (0, 0)),
            pl.BlockSpec((dout, dout), lambda k: (0, 0)),
            pl.BlockSpec((1, 4), lambda k: (0, 0)),
        ],
        out_specs=pl.BlockSpec((2, nb, 100, 100), lambda k: (0, k, 0, 0)),
        out_shape=jax.ShapeDtypeStruct((2, 100, 100, 100), jnp.float32),
    )(q, g2, dinv, b2, matrix, wlbl)


def kernel(x, edge_index, W_t1, b_t1, W_t2, b_t2, W_c1, b_c1, W_c2, b_c2,
           matrix, W_lin, b_lin):
    eidx = edge_index
    eidx6 = edge_index.reshape(2, NC, NS, NCHUNK1, 1, CH1)

    deg = _sc_deg(eidx)
    deg3 = deg[:, :N, None]

    g1, dinv = _tc_proj(W_t1, W_t2, (b_t1 + b_t2)[None, :], W_c1, x,
                        deg3)

    p1 = _sc_conv1(eidx6, g1)
    g2 = _tc_mid(p1, g1, dinv, b_c1[None, :], W_c2)

    p2 = _sc_conv(eidx, g2, g2.shape[1])

    wlbl = jnp.concatenate([W_lin, b_lin])[None, :]
    o = _tc_head(p2, g2, dinv, b_c2[None, :], matrix, wlbl)

    return o.reshape(2, -1).T


# speedup vs baseline: 33.7806x; 1.0340x over previous
import functools

import jax
import jax.numpy as jnp
from jax import lax
from jax.experimental import pallas as pl
from jax.experimental.pallas import tpu as pltpu
from jax.experimental.pallas import tpu_sc as plsc

N = 10000
E = 320000
NPAD = 10240
NC, NS = 2, 16
NW = NC * NS
CH = 128
TOTCH = E // CH
NCHUNK = TOTCH // NW
NEXTRA = TOTCH - NCHUNK * NW
RPT = NPAD // NS

_PREC = jax.lax.Precision.HIGHEST


NBUF = 4


def _sc_deg_body(eidx, deg_out, acc, idx_v, ones_v, zb, isems, ssems):
    c = lax.axis_index("c")
    s = lax.axis_index("s")
    w = c * NS + s
    for i in range(RPT // 16):
        zb[pl.ds(i * 16, 16)] = jnp.zeros((16,), jnp.float32)
    for i in range(CH // 16):
        ones_v[pl.ds(i * 16, 16)] = jnp.ones((16,), jnp.float32)
    pltpu.sync_copy(zb, acc.at[pl.ds(s * RPT, RPT)])
    plsc.subcore_barrier()

    def idx_issue(j, ib):
        pltpu.async_copy(eidx.at[:, pl.ds((j * NW + w) * CH, CH)],
                         idx_v.at[ib], isems.at[ib])

    def idx_wait(j, ib):
        pltpu.make_async_copy(eidx.at[:, pl.ds((j * NW + w) * CH, CH)],
                              idx_v.at[ib], isems.at[ib]).wait()

    def visit(j, b, ib, ibp, drain, refill):
        idx_wait(j, ib)
        if drain:
            pltpu.make_async_copy(ones_v, acc.at[idx_v.at[ib, 1]],
                                  ssems.at[b]).wait()
        pltpu.async_copy(ones_v, acc.at[idx_v.at[ib, 1]], ssems.at[b],
                         add=True)
        if refill:
            idx_issue(j + NBUF, ibp)

    for j in range(NBUF):
        idx_issue(j, j)
    RND = 2 * NBUF
    for j in range(NBUF):
        visit(j, j % NBUF, j % RND, (j + NBUF) % RND,
              drain=False, refill=True)

    nfull = (NCHUNK - 2 * NBUF) // RND

    def stepw(jj, carry):
        j0 = NBUF + jj * RND
        for k in range(RND):
            j = j0 + k
            visit(j, (NBUF + k) % NBUF, (NBUF + k) % RND, k % RND,
                  drain=True, refill=True)
        return carry

    lax.fori_loop(0, nfull, stepw, 0)
    for j in range(NBUF + nfull * RND, NCHUNK):
        visit(j, j % NBUF, j % RND, (j + NBUF) % RND,
              drain=True, refill=(j + NBUF < NCHUNK))
    for b in range(NBUF):
        pltpu.make_async_copy(ones_v, acc.at[idx_v.at[0, 1]],
                              ssems.at[b]).wait()

    @pl.when(w < NEXTRA)
    def _():
        pltpu.sync_copy(eidx.at[:, pl.ds((NCHUNK * NW + w) * CH, CH)],
                        idx_v.at[0])
        pltpu.sync_copy(ones_v, acc.at[idx_v.at[0, 1]], add=True)

    plsc.subcore_barrier()
    pltpu.sync_copy(acc.at[pl.ds(s * RPT, RPT)],
                    deg_out.at[c, pl.ds(s * RPT, RPT)])


def _sc_deg(eidx):
    mesh = plsc.VectorSubcoreMesh(core_axis_name="c", subcore_axis_name="s")
    return pl.kernel(
        _sc_deg_body,
        out_type=jax.ShapeDtypeStruct((NC, NPAD), jnp.float32),
        mesh=mesh,
        scratch_types=[
            pltpu.VMEM_SHARED((NPAD,), jnp.float32),
            pltpu.VMEM((2 * NBUF, 2, CH), jnp.int32),
            pltpu.VMEM((CH,), jnp.float32),
            pltpu.VMEM((RPT,), jnp.float32),
            pltpu.SemaphoreType.DMA((2 * NBUF,)),
            pltpu.SemaphoreType.DMA((NBUF,)),
        ],
    )(eidx)


CH1 = 80
NCHUNK1 = (E // NW) // CH1


def _sc_conv1_body(D, NB, eidx, g, out, acc, sidx_v, didx_v, rows_v, zb,
                   isems, dsems, gsems):
    c = lax.axis_index("c")
    s = lax.axis_index("s")
    for r in range(16):
        for q in range(D // 16):
            zb[r, pl.ds(q * 16, 16)] = jnp.zeros((16,), jnp.float32)

    def zstep(j, carry):
        pltpu.sync_copy(zb, acc.at[pl.ds(s * RPT + j * 16, 16)])
        return carry

    lax.fori_loop(0, RPT // 16, zstep, 0)
    plsc.subcore_barrier()

    def idx_issue(j, ib):
        pltpu.async_copy(eidx.at[0, c, s, j], sidx_v.at[ib], isems.at[ib])
        pltpu.async_copy(eidx.at[1, c, s, j], didx_v.at[ib], dsems.at[ib])

    def idx_wait(j, ib):
        pltpu.make_async_copy(eidx.at[0, c, s, j], sidx_v.at[ib],
                              isems.at[ib]).wait()
        pltpu.make_async_copy(eidx.at[1, c, s, j], didx_v.at[ib],
                              dsems.at[ib]).wait()

    def gat_issue(ib, b):
        pltpu.async_copy(g.at[sidx_v.at[ib, 0]], rows_v.at[b], gsems.at[b])

    def gat_wait(ib, b):
        pltpu.make_async_copy(g.at[sidx_v.at[ib, 0]], rows_v.at[b],
                              gsems.at[b]).wait()

    for j in range(NB):
        idx_issue(j, j)
    for j in range(NB):
        idx_wait(j, j)
        gat_issue(j, j)
    for j in range(NB, 2 * NB):
        idx_issue(j, j)

    def visit(j, b, ib, ib2, refill, advance):
        gat_wait(ib, b)
        pltpu.sync_copy(rows_v.at[b], acc.at[didx_v.at[ib, 0]], add=True)
        if advance:
            idx_wait(j + NB, ib2)
            gat_issue(ib2, b)
        if refill:
            idx_issue(j + 2 * NB, ib)

    RND = 2 * NB
    nfull = (NCHUNK1 - RND) // RND

    def step(jj, carry):
        j0 = jj * RND
        for k in range(RND):
            visit(j0 + k, k % NB, k, (k + NB) % RND, True, True)
        return carry

    lax.fori_loop(0, nfull, step, 0)
    for j in range(nfull * RND, NCHUNK1):
        visit(j, j % NB, j % RND, (j + NB) % RND,
              refill=(j + 2 * NB < NCHUNK1),
              advance=(j + NB < NCHUNK1))
    plsc.subcore_barrier()
    pltpu.sync_copy(acc.at[pl.ds(s * RPT, RPT)],
                    out.at[c, pl.ds(s * RPT, RPT)])


def _sc_conv1(eidx6, g):
    D, NB = 128, 4
    mesh = plsc.VectorSubcoreMesh(core_axis_name="c", subcore_axis_name="s")
    return pl.kernel(
        functools.partial(_sc_conv1_body, D, NB),
        out_type=jax.ShapeDtypeStruct((NC, NPAD, D), jnp.float32),
        mesh=mesh,
        compiler_params=pltpu.CompilerParams(use_tc_tiling_on_sc=True),
        scratch_types=[
            pltpu.VMEM_SHARED((NPAD, D), jnp.float32),
            pltpu.VMEM((2 * NB, 1, CH1), jnp.int32),
            pltpu.VMEM((2 * NB, 1, CH1), jnp.int32),
            pltpu.VMEM((NB, CH1, D), jnp.float32),
            pltpu.VMEM((16, D), jnp.float32),
            pltpu.SemaphoreType.DMA((2 * NB,)),
            pltpu.SemaphoreType.DMA((2 * NB,)),
            pltpu.SemaphoreType.DMA((NB,)),
        ],
    )(eidx6, g)


def _sc_conv_body(D, NB, eidx, g, out, acc, idx_v, rows_v, zb, isems, gsems):
    c = lax.axis_index("c")
    s = lax.axis_index("s")
    w = c * NS + s
    for r in range(16):
        for q in range(D // 16):
            zb[r, pl.ds(q * 16, 16)] = jnp.zeros((16,), jnp.float32)

    def zstep(j, carry):
        pltpu.sync_copy(zb, acc.at[pl.ds(s * RPT + j * 16, 16)])
        return carry

    lax.fori_loop(0, RPT // 16, zstep, 0)
    plsc.subcore_barrier()

    def idx_issue(j, ib):
        pltpu.async_copy(eidx.at[:, pl.ds((j * NW + w) * CH, CH)],
                         idx_v.at[ib], isems.at[ib])

    def idx_wait(j, ib):
        pltpu.make_async_copy(eidx.at[:, pl.ds((j * NW + w) * CH, CH)],
                              idx_v.at[ib], isems.at[ib]).wait()

    def gat_issue(ib, b):
        pltpu.async_copy(g.at[idx_v.at[ib, 0]], rows_v.at[b], gsems.at[b])

    def gat_wait(ib, b):
        pltpu.make_async_copy(g.at[idx_v.at[ib, 0]], rows_v.at[b],
                              gsems.at[b]).wait()

    for j in range(NB):
        idx_issue(j, j)
    for j in range(NB):
        idx_wait(j, j)
        gat_issue(j, j)
    for j in range(NB, 2 * NB):
        idx_issue(j, j)

    def visit(j, b, ib, ib2, refill, advance):
        gat_wait(ib, b)
        pltpu.sync_copy(rows_v.at[b], acc.at[idx_v.at[ib, 1]], add=True)
        if advance:
            idx_wait(j + NB, ib2)
            gat_issue(ib2, b)
        if refill:
            idx_issue(j + 2 * NB, ib)

    RND = 2 * NB
    nfull = (NCHUNK - RND) // RND

    def step(jj, carry):
        j0 = jj * RND
        for k in range(RND):
            visit(j0 + k, k % NB, k, (k + NB) % RND, True, True)
        return carry

    lax.fori_loop(0, nfull, step, 0)
    for j in range(nfull * RND, NCHUNK):
        visit(j, j % NB, j % RND, (j + NB) % RND,
              refill=(j + 2 * NB < NCHUNK),
              advance=(j + NB < NCHUNK))

    @pl.when(w < NEXTRA)
    def _():
        pltpu.sync_copy(eidx.at[:, pl.ds((NCHUNK * NW + w) * CH, CH)],
                        idx_v.at[0])
        pltpu.async_copy(g.at[idx_v.at[0, 0]], rows_v.at[0],
                         gsems.at[0]).wait()
        pltpu.sync_copy(rows_v.at[0], acc.at[idx_v.at[0, 1]], add=True)

    plsc.subcore_barrier()
    pltpu.sync_copy(acc.at[pl.ds(s * RPT, RPT)],
                    out.at[c, pl.ds(s * RPT, RPT)])


def _sc_conv(eidx, g, D):
    NB = 2 if D == 128 else 4
    mesh = plsc.VectorSubcoreMesh(core_axis_name="c", subcore_axis_name="s")
    return pl.kernel(
        functools.partial(_sc_conv_body, D, NB),
        out_type=jax.ShapeDtypeStruct((NC, NPAD, D), jnp.float32),
        mesh=mesh,
        compiler_params=pltpu.CompilerParams(
            use_tc_tiling_on_sc=(D % 128 == 0)),
        scratch_types=[
            pltpu.VMEM_SHARED((NPAD, D), jnp.float32),
            pltpu.VMEM((2 * NB, 2, CH), jnp.int32),
            pltpu.VMEM((NB, CH, D), jnp.float32),
            pltpu.VMEM((16, D), jnp.float32),
            pltpu.SemaphoreType.DMA((2 * NB,)),
            pltpu.SemaphoreType.DMA((NB,)),
        ],
    )(eidx, g)


def _tc_proj_body(wt1_ref, wt2_ref, b12_ref, wc1_ref, x_ref, deg_ref,
                  g1_ref, dinv_ref, wbig_ref, bias1_ref):
    @pl.when(pl.program_id(0) == 0)
    def _():
        wc1 = wc1_ref[...]
        wbig_ref[0:128, :] = jnp.dot(wt1_ref[...], wc1,
                                     preferred_element_type=jnp.float32,
                                     precision=_PREC)
        wbig_ref[128:896, :] = jnp.dot(wt2_ref[...], wc1,
                                       preferred_element_type=jnp.float32,
                                       precision=_PREC)
        bias1_ref[...] = jnp.dot(b12_ref[...], wc1,
                                 preferred_element_type=jnp.float32,
                                 precision=_PREC)

    d = deg_ref[0, :, 0] + deg_ref[1, :, 0] + 1.0
    dinv = lax.rsqrt(d)
    hw = jnp.dot(x_ref[...], wbig_ref[...],
                 preferred_element_type=jnp.float32)
    hw = hw + bias1_ref[...]
    g1_ref[...] = hw * dinv[:, None]
    dinv_ref[...] = dinv[:, None]


def _tc_proj(wt1, wt2, b12, wc1, x, deg, bm=1000):
    nm = N // bm
    hid = wc1.shape[1]
    return pl.pallas_call(
        _tc_proj_body,
        grid=(nm,),
        in_specs=[
            pl.BlockSpec(wt1.shape, lambda m: (0, 0)),
            pl.BlockSpec(wt2.shape, lambda m: (0, 0)),
            pl.BlockSpec(b12.shape, lambda m: (0, 0)),
            pl.BlockSpec(wc1.shape, lambda m: (0, 0)),
            pl.BlockSpec((bm, x.shape[1]), lambda m: (m, 0)),
            pl.BlockSpec((NC, bm, 1), lambda m: (0, m, 0)),
        ],
        out_specs=[
            pl.BlockSpec((bm, hid), lambda m: (m, 0)),
            pl.BlockSpec((bm, 1), lambda m: (m, 0)),
        ],
        out_shape=[
            jax.ShapeDtypeStruct((N, hid), jnp.float32),
            jax.ShapeDtypeStruct((N, 1), jnp.float32),
        ],
        scratch_shapes=[
            pltpu.VMEM((896, 128), jnp.float32),
            pltpu.VMEM((1, 128), jnp.float32),
        ],
    )(wt1, wt2, b12, wc1, x, deg)


def _tc_mid_body(p_ref, g1_ref, dinv_ref, b1_ref, w2_ref, g2_ref):
    ssum = p_ref[0] + p_ref[1] + g1_ref[...]
    h1 = jnp.maximum(ssum * dinv_ref[...] + b1_ref[...], 0.0)
    hw2 = jnp.dot(h1, w2_ref[...], preferred_element_type=jnp.float32)
    g2_ref[...] = hw2 * dinv_ref[...]


def _tc_mid(p, g1, dinv, b1, w2, bm=1000):
    nm = N // bm
    hid = g1.shape[1]
    dout = w2.shape[1]
    return pl.pallas_call(
        _tc_mid_body,
        grid=(nm,),
        in_specs=[
            pl.BlockSpec((NC, bm, hid), lambda m: (0, m, 0)),
            pl.BlockSpec((bm, hid), lambda m: (m, 0)),
            pl.BlockSpec((bm, 1), lambda m: (m, 0)),
            pl.BlockSpec((1, hid), lambda m: (0, 0)),
            pl.BlockSpec((hid, dout), lambda m: (0, 0)),
        ],
        out_specs=pl.BlockSpec((bm, dout), lambda m: (m, 0)),
        out_shape=jax.ShapeDtypeStruct((N, dout), jnp.float32),
    )(p, g1, dinv, b1, w2)


def _tc_head_body(nb, q_ref, g2_ref, dinv_ref, b2_ref, m_ref, wl_ref, o_ref):
    w0 = wl_ref[0, 0]
    w1 = wl_ref[0, 1]
    c0 = wl_ref[0, 2]
    c1 = wl_ref[0, 3]
    for i in range(nb):
        r = pl.ds(i * 100, 100)
        q = q_ref[0, r, :] + q_ref[1, r, :] + g2_ref[r, :]
        h2 = q * dinv_ref[r, :] + b2_ref[...]
        t = jnp.dot(h2, m_ref[...], preferred_element_type=jnp.float32)
        s = lax.dot_general(t, h2, (((1,), (1,)), ((), ())),
                            preferred_element_type=jnp.float32)
        o_ref[0, i] = s * w0 + c0
        o_ref[1, i] = s * w1 + c1


def _tc_head(q, g2, dinv, b2, matrix, wlbl, nb=4):
    bm = nb * 100
    ng = 100 // nb
    dout = matrix.shape[1]
    return pl.pallas_call(
        functools.partial(_tc_head_body, nb),
        grid=(ng,),
        in_specs=[
            pl.BlockSpec((NC, bm, dout), lambda k: (0, k, 0)),
            pl.BlockSpec((bm, dout), lambda k: (k, 0)),
            pl.BlockSpec((bm, 1), lambda k: (k, 0)),
            pl.BlockSpec((1, dout), lambda k: (0, 0)),
            pl.BlockSpec((dout, dout), lambda k: (0, 0)),
            pl.BlockSpec((1, 4), lambda k: (0, 0)),
        ],
        out_specs=pl.BlockSpec((2, nb, 100, 100), lambda k: (0, k, 0, 0)),
        out_shape=jax.ShapeDtypeStruct((2, 100, 100, 100), jnp.float32),
    )(q, g2, dinv, b2, matrix, wlbl)


def kernel(x, edge_index, W_t1, b_t1, W_t2, b_t2, W_c1, b_c1, W_c2, b_c2,
           matrix, W_lin, b_lin):
    eidx = edge_index
    eidx6 = edge_index.reshape(2, NC, NS, NCHUNK1, 1, CH1)

    deg = _sc_deg(eidx)
    deg3 = deg[:, :N, None]

    g1, dinv = _tc_proj(W_t1, W_t2, (b_t1 + b_t2)[None, :], W_c1, x,
                        deg3)

    p1 = _sc_conv1(eidx6, g1)
    g2 = _tc_mid(p1, g1, dinv, b_c1[None, :], W_c2)

    p2 = _sc_conv(eidx, g2, g2.shape[1])

    wlbl = jnp.concatenate([W_lin, b_lin])[None, :]
    o = _tc_head(p2, g2, dinv, b_c2[None, :], matrix, wlbl)

    return o.reshape(2, -1).T
